# baseline (speedup 1.0000x reference)
import functools

import jax
import jax.numpy as jnp
import numpy as np
from jax import lax
from jax.experimental import pallas as pl
from jax.experimental.pallas import tpu as pltpu
from jax.experimental.pallas import tpu_sc as plsc

W_EARTH = 7.2921151467e-05
GMST0 = 1.7321

NUM_STATIONS = 4096
LANES = 16
NUM_CORES = 2
NUM_SUBCORES = 16
NUM_WORKERS = NUM_CORES * NUM_SUBCORES

TABLE_STEPS = 4096
STEP = 2.0 * np.pi / TABLE_STEPS
TABLE_LEN = 4160
_angles = GMST0 + np.arange(TABLE_LEN, dtype=np.float64) * STEP
_TRIG_TAB = np.concatenate(
    [np.cos(_angles), np.sin(_angles)]
).astype(np.float32)

CHUNK = 2048
GROUPS = CHUNK // LANES
UNROLL = 16


def _sc_ground_stations(t_hbm, idx_hbm, st_hbm, tab_hbm,
                        xt_hbm, yt_hbm, z_hbm, vx_hbm, vy_hbm,
                        st_v, tab_v, t_v, i_v, stage, in_sem, out_sem):
    n = t_hbm.shape[0]
    elems = n // NUM_WORKERS
    nchunk = elems // CHUNK

    cid = lax.axis_index("c")
    sid = lax.axis_index("s")
    wid = sid * NUM_CORES + cid
    base = wid * elems

    pltpu.sync_copy(st_hbm, st_v)
    pltpu.sync_copy(tab_hbm, tab_v)

    out_hbms = (xt_hbm, yt_hbm, z_hbm, vx_hbm, vy_hbm)

    def in_copy(cur, b):
        off = base + cur * CHUNK
        pltpu.async_copy(t_hbm.at[pl.ds(off, CHUNK)], t_v[b], in_sem[b])
        pltpu.async_copy(idx_hbm.at[pl.ds(off, CHUNK)], i_v[b], in_sem[b])

    def wait_in(b):
        pltpu.make_async_copy(t_hbm.at[pl.ds(0, CHUNK)], t_v[b],
                              in_sem[b]).wait()
        pltpu.make_async_copy(idx_hbm.at[pl.ds(0, CHUNK)], i_v[b],
                              in_sem[b]).wait()

    def out_copy(cur, b):
        off = base + cur * CHUNK
        for o, hbm in enumerate(out_hbms):
            pltpu.async_copy(stage[b][o], hbm.at[pl.ds(off, CHUNK)],
                             out_sem[b])

    def wait_out(b):
        for o, hbm in enumerate(out_hbms):
            pltpu.make_async_copy(stage[b][o], hbm.at[pl.ds(0, CHUNK)],
                                  out_sem[b]).wait()

    def compute(b):
        tb, ib = t_v[b], i_v[b]
        xt_s, yt_s, z_s, vx_s, vy_s = stage[b]

        @plsc.parallel_loop(0, GROUPS, unroll=UNROLL)
        def grp(g):
            sl = pl.ds(g * LANES, LANES)
            t = tb[sl]
            ix = ib[sl]
            u = t * W_EARTH
            k = (u * (1.0 / STEP) + 0.5).astype(jnp.int32)
            c = plsc.load_gather(tab_v, [k])
            s = plsc.load_gather(tab_v, [k + TABLE_LEN])
            x = plsc.load_gather(st_v, [ix])
            y = plsc.load_gather(st_v, [ix + NUM_STATIONS])
            z = plsc.load_gather(st_v, [ix + 2 * NUM_STATIONS])
            xt = x * c - y * s
            yt = x * s + y * c
            xt_s[sl] = xt
            yt_s[sl] = yt
            z_s[sl] = z
            vx_s[sl] = yt * np.float32(-W_EARTH)
            vy_s[sl] = xt * np.float32(W_EARTH)

    in_copy(0, 0)
    in_copy(1, 1)

    @pl.loop(0, nchunk, step=2)
    def outer(ci):
        for b in range(2):
            cur = ci + b
            wait_in(b)

            @pl.when(cur >= 2)
            def _():
                wait_out(b)

            compute(b)
            out_copy(cur, b)

            @pl.when(cur + 2 < nchunk)
            def _():
                in_copy(cur + 2, b)

    wait_out(0)
    wait_out(1)


def kernel(t_tai, station_indices, stations_ecef):
    n = t_tai.shape[0]
    st_flat = stations_ecef.T.reshape(-1)
    tab = jnp.asarray(_TRIG_TAB)

    mesh = plsc.VectorSubcoreMesh(
        core_axis_name="c", subcore_axis_name="s",
        num_cores=NUM_CORES, num_subcores=NUM_SUBCORES)

    plane = jax.ShapeDtypeStruct((n,), jnp.float32)
    fbuf = pltpu.VMEM((CHUNK,), jnp.float32)
    call = functools.partial(
        pl.kernel,
        out_type=[plane] * 5,
        mesh=mesh,
        compiler_params=pltpu.CompilerParams(needs_layout_passes=False),
        scratch_types=[
            pltpu.VMEM((3 * NUM_STATIONS,), jnp.float32),
            pltpu.VMEM((2 * TABLE_LEN,), jnp.float32),
            [fbuf, fbuf],
            [pltpu.VMEM((CHUNK,), jnp.int32)] * 2,
            [[fbuf] * 5, [fbuf] * 5],
            [pltpu.SemaphoreType.DMA] * 2,
            [pltpu.SemaphoreType.DMA] * 2,
        ],
    )(_sc_ground_stations)

    xt, yt, z, vx, vy = call(t_tai, station_indices, st_flat, tab)
    pos = jnp.stack([xt, yt, z], axis=1)
    vel = jnp.stack([vx, vy, jnp.zeros_like(xt)], axis=1)
    return pos, vel


# speedup vs baseline: 1.0679x; 1.0679x over previous
import functools

import jax
import jax.numpy as jnp
import numpy as np
from jax import lax
from jax.experimental import pallas as pl
from jax.experimental.pallas import tpu as pltpu
from jax.experimental.pallas import tpu_sc as plsc

W_EARTH = 7.2921151467e-05
GMST0 = 1.7321

NUM_STATIONS = 4096
LANES = 16
NUM_CORES = 2
NUM_SUBCORES = 16
NUM_WORKERS = NUM_CORES * NUM_SUBCORES

TABLE_STEPS = 4096
STEP = 2.0 * np.pi / TABLE_STEPS
TABLE_LEN = 4160
_angles = GMST0 + np.arange(TABLE_LEN, dtype=np.float64) * STEP
_TRIG_TAB = np.concatenate(
    [np.cos(_angles), np.sin(_angles)]
).astype(np.float32)

CHUNK = 4096
GROUPS = CHUNK // LANES
UNROLL = 16


def _sc_ground_stations(t_hbm, idx_hbm, st_hbm, tab_hbm,
                        xt_hbm, yt_hbm, z_hbm, vx_hbm, vy_hbm,
                        st_v, tab_v, t_v, i_v, stage, in_sem, out_sem):
    n = t_hbm.shape[0]
    elems = n // NUM_WORKERS
    nchunk = elems // CHUNK

    cid = lax.axis_index("c")
    sid = lax.axis_index("s")
    wid = sid * NUM_CORES + cid
    base = wid * elems

    pltpu.sync_copy(st_hbm, st_v)
    pltpu.sync_copy(tab_hbm, tab_v)

    out_hbms = (xt_hbm, yt_hbm, z_hbm, vx_hbm, vy_hbm)

    def in_copy(cur, b):
        off = base + cur * CHUNK
        pltpu.async_copy(t_hbm.at[pl.ds(off, CHUNK)], t_v[b], in_sem[b])
        pltpu.async_copy(idx_hbm.at[pl.ds(off, CHUNK)], i_v[b], in_sem[b])

    def wait_in(b):
        pltpu.make_async_copy(t_hbm.at[pl.ds(0, CHUNK)], t_v[b],
                              in_sem[b]).wait()
        pltpu.make_async_copy(idx_hbm.at[pl.ds(0, CHUNK)], i_v[b],
                              in_sem[b]).wait()

    def out_copy(cur, b):
        off = base + cur * CHUNK
        for o, hbm in enumerate(out_hbms):
            pltpu.async_copy(stage[b][o], hbm.at[pl.ds(off, CHUNK)],
                             out_sem[b])

    def wait_out(b):
        for o, hbm in enumerate(out_hbms):
            pltpu.make_async_copy(stage[b][o], hbm.at[pl.ds(0, CHUNK)],
                                  out_sem[b]).wait()

    def compute(b):
        tb, ib = t_v[b], i_v[b]
        xt_s, yt_s, z_s, vx_s, vy_s = stage[b]

        @plsc.parallel_loop(0, GROUPS, unroll=UNROLL)
        def grp(g):
            sl = pl.ds(g * LANES, LANES)
            t = tb[sl]
            ix = ib[sl]
            u = t * W_EARTH
            k = (u * (1.0 / STEP) + 0.5).astype(jnp.int32)
            c = plsc.load_gather(tab_v, [k])
            s = plsc.load_gather(tab_v, [k + TABLE_LEN])
            x = plsc.load_gather(st_v, [ix])
            y = plsc.load_gather(st_v, [ix + NUM_STATIONS])
            z = plsc.load_gather(st_v, [ix + 2 * NUM_STATIONS])
            xt = x * c - y * s
            yt = x * s + y * c
            xt_s[sl] = xt
            yt_s[sl] = yt
            z_s[sl] = z
            vx_s[sl] = yt * np.float32(-W_EARTH)
            vy_s[sl] = xt * np.float32(W_EARTH)

    in_copy(0, 0)
    in_copy(1, 1)

    @pl.loop(0, nchunk, step=2)
    def outer(ci):
        for b in range(2):
            cur = ci + b
            wait_in(b)

            @pl.when(cur >= 2)
            def _():
                wait_out(b)

            compute(b)
            out_copy(cur, b)

            @pl.when(cur + 2 < nchunk)
            def _():
                in_copy(cur + 2, b)

    wait_out(0)
    wait_out(1)


def kernel(t_tai, station_indices, stations_ecef):
    n = t_tai.shape[0]
    st_flat = stations_ecef.T.reshape(-1)
    tab = jnp.asarray(_TRIG_TAB)

    mesh = plsc.VectorSubcoreMesh(
        core_axis_name="c", subcore_axis_name="s",
        num_cores=NUM_CORES, num_subcores=NUM_SUBCORES)

    plane = jax.ShapeDtypeStruct((n,), jnp.float32)
    fbuf = pltpu.VMEM((CHUNK,), jnp.float32)
    call = functools.partial(
        pl.kernel,
        out_type=[plane] * 5,
        mesh=mesh,
        compiler_params=pltpu.CompilerParams(needs_layout_passes=False),
        scratch_types=[
            pltpu.VMEM((3 * NUM_STATIONS,), jnp.float32),
            pltpu.VMEM((2 * TABLE_LEN,), jnp.float32),
            [fbuf, fbuf],
            [pltpu.VMEM((CHUNK,), jnp.int32)] * 2,
            [[fbuf] * 5, [fbuf] * 5],
            [pltpu.SemaphoreType.DMA] * 2,
            [pltpu.SemaphoreType.DMA] * 2,
        ],
    )(_sc_ground_stations)

    xt, yt, z, vx, vy = call(t_tai, station_indices, st_flat, tab)
    pos = jnp.stack([xt, yt, z], axis=1)
    vel = jnp.stack([vx, vy, jnp.zeros_like(xt)], axis=1)
    return pos, vel
